# baseline (speedup 1.0000x reference)
import functools

import jax
import jax.numpy as jnp
from jax import lax
from jax.experimental import pallas as pl
from jax.experimental.pallas import tpu as pltpu
from jax.experimental.pallas import tpu_sc as plsc

B = 1024
L = 50
EMB = 64
VOCAB = 100000

NC = 2
NS = 16
NW = NC * NS
BPW = B // NW
IDW = BPW * L
CHUNK = 128
NCH = (IDW + CHUNK - 1) // CHUNK

_mesh = plsc.VectorSubcoreMesh(core_axis_name="c", subcore_axis_name="s")


@functools.partial(
    pl.kernel,
    mesh=_mesh,
    out_type=jax.ShapeDtypeStruct((B, EMB), jnp.float32),
    scratch_types=[
        pltpu.VMEM((IDW,), jnp.int32),
        pltpu.VMEM((IDW, EMB), jnp.float32),
        pltpu.VMEM((BPW, EMB), jnp.float32),
        pltpu.SemaphoreType.DMA,
    ],
    compiler_params=pltpu.CompilerParams(use_tc_tiling_on_sc=False),
)
def _pool_sc(idx_hbm, table_hbm, out_hbm, idx_v, rows_v, out_v, sem):
    wid = lax.axis_index("s") * NC + lax.axis_index("c")
    base = wid * BPW
    pltpu.sync_copy(idx_hbm.at[pl.ds(base * L, IDW)], idx_v)

    copies = []
    for c in range(NCH):
        n = min(CHUNK, IDW - c * CHUNK)
        copies.append(
            pltpu.async_copy(
                table_hbm.at[idx_v.at[pl.ds(c * CHUNK, n)]],
                rows_v.at[pl.ds(c * CHUNK, n)],
                sem,
            )
        )
    for cp in copies:
        cp.wait()

    def body(i, _):
        rbase = i * L
        for c in range(EMB // 16):
            sl = pl.ds(c * 16, 16)
            acc = rows_v[rbase, sl]
            for j in range(1, L):
                acc = acc + rows_v[rbase + j, sl]
            out_v[i, sl] = acc * (1.0 / L)
        return 0

    lax.fori_loop(0, BPW, body, 0)
    pltpu.sync_copy(out_v, out_hbm.at[pl.ds(base, BPW)])


NBLK = 1024
_GRID = (VOCAB + NBLK - 1) // NBLK
_TAIL = VOCAB - (_GRID - 1) * NBLK
NBUF = 8


def _out_copy(o_hbm, obuf, sems, step, slot):
    return pltpu.make_async_copy(
        obuf.at[slot],
        o_hbm.at[:, pl.ds(step * NBLK, NBLK)],
        sems.at[slot],
    )


def _proj_tc(p_ref, w_ref, b_ref, o_hbm, obuf, tailbuf, sems, tailsem):
    i = pl.program_id(0)
    slot = lax.rem(i, NBUF)

    @pl.when(i >= NBUF)
    def _():
        _out_copy(o_hbm, obuf, sems, i - NBUF, slot).wait()

    blk = (
        jnp.dot(p_ref[...], w_ref[...], preferred_element_type=jnp.float32)
        + b_ref[...]
    )

    @pl.when(jnp.logical_and(i < _GRID - 1, lax.rem(i, 2) == 0))
    def _():
        obuf[slot] = blk
        _out_copy(o_hbm, obuf, sems, i, slot).start(priority=0)

    @pl.when(jnp.logical_and(i < _GRID - 1, lax.rem(i, 2) == 1))
    def _():
        obuf[slot] = blk
        _out_copy(o_hbm, obuf, sems, i, slot).start(priority=1)

    @pl.when(i == _GRID - 1)
    def _():
        tailbuf[...] = blk[:, :_TAIL]
        tail = pltpu.make_async_copy(
            tailbuf,
            o_hbm.at[:, pl.ds((_GRID - 1) * NBLK, _TAIL)],
            tailsem,
        )
        tail.start(priority=0)
        for back in range(1, NBUF):
            st = _GRID - 1 - back
            _out_copy(o_hbm, obuf, sems, st, lax.rem(st, NBUF)).wait()
        tail.wait()


def kernel(inputs, emb_table, W, b):
    pooled = _pool_sc(inputs.reshape(-1), emb_table)
    return pl.pallas_call(
        _proj_tc,
        grid=(_GRID,),
        in_specs=[
            pl.BlockSpec((B, EMB), lambda i: (0, 0)),
            pl.BlockSpec((EMB, NBLK), lambda i: (0, i)),
            pl.BlockSpec((1, NBLK), lambda i: (0, i)),
        ],
        out_specs=pl.BlockSpec(memory_space=pl.ANY),
        out_shape=jax.ShapeDtypeStruct((B, VOCAB), jnp.float32),
        scratch_shapes=[
            pltpu.VMEM((NBUF, B, NBLK), jnp.float32),
            pltpu.VMEM((B, _TAIL), jnp.float32),
            pltpu.SemaphoreType.DMA((NBUF,)),
            pltpu.SemaphoreType.DMA,
        ],
    )(pooled, W, b.reshape(1, VOCAB))


# speedup vs baseline: 1.0169x; 1.0169x over previous
import functools

import jax
import jax.numpy as jnp
from jax import lax
from jax.experimental import pallas as pl
from jax.experimental.pallas import tpu as pltpu
from jax.experimental.pallas import tpu_sc as plsc

B = 1024
L = 50
EMB = 64
VOCAB = 100000

NC = 2
NS = 16
NW = NC * NS
BPW = B // NW
IDW = BPW * L
CHUNK = 128
NCH = (IDW + CHUNK - 1) // CHUNK

_mesh = plsc.VectorSubcoreMesh(core_axis_name="c", subcore_axis_name="s")


@functools.partial(
    pl.kernel,
    mesh=_mesh,
    out_type=jax.ShapeDtypeStruct((B, EMB), jnp.float32),
    scratch_types=[
        pltpu.VMEM((IDW,), jnp.int32),
        pltpu.VMEM((IDW, EMB), jnp.float32),
        pltpu.VMEM((BPW, EMB), jnp.float32),
        pltpu.SemaphoreType.DMA,
    ],
    compiler_params=pltpu.CompilerParams(use_tc_tiling_on_sc=False),
)
def _pool_sc(idx_hbm, table_hbm, out_hbm, idx_v, rows_v, out_v, sem):
    wid = lax.axis_index("s") * NC + lax.axis_index("c")
    base = wid * BPW
    pltpu.sync_copy(idx_hbm.at[pl.ds(base * L, IDW)], idx_v)

    copies = []
    for c in range(NCH):
        n = min(CHUNK, IDW - c * CHUNK)
        copies.append(
            pltpu.async_copy(
                table_hbm.at[idx_v.at[pl.ds(c * CHUNK, n)]],
                rows_v.at[pl.ds(c * CHUNK, n)],
                sem,
            )
        )
    for cp in copies:
        cp.wait()

    def body(i, _):
        rbase = i * L
        for c in range(EMB // 16):
            sl = pl.ds(c * 16, 16)
            acc = rows_v[rbase, sl]
            for j in range(1, L):
                acc = acc + rows_v[rbase + j, sl]
            out_v[i, sl] = acc * (1.0 / L)
        return 0

    lax.fori_loop(0, BPW, body, 0)
    pltpu.sync_copy(out_v, out_hbm.at[pl.ds(base, BPW)])


NBLK = 2048
_GRID = (VOCAB + NBLK - 1) // NBLK
_TAIL = VOCAB - (_GRID - 1) * NBLK
NBUF = 4


def _out_copy(o_hbm, obuf, sems, step, slot):
    return pltpu.make_async_copy(
        obuf.at[slot],
        o_hbm.at[:, pl.ds(step * NBLK, NBLK)],
        sems.at[slot],
    )


def _proj_tc(p_ref, w_ref, b_ref, o_hbm, obuf, tailbuf, sems, tailsem):
    i = pl.program_id(0)
    slot = lax.rem(i, NBUF)

    @pl.when(i >= NBUF)
    def _():
        _out_copy(o_hbm, obuf, sems, i - NBUF, slot).wait()

    blk = (
        jnp.dot(p_ref[...], w_ref[...], preferred_element_type=jnp.float32)
        + b_ref[...]
    )

    @pl.when(jnp.logical_and(i < _GRID - 1, lax.rem(i, 2) == 0))
    def _():
        obuf[slot] = blk
        _out_copy(o_hbm, obuf, sems, i, slot).start(priority=0)

    @pl.when(jnp.logical_and(i < _GRID - 1, lax.rem(i, 2) == 1))
    def _():
        obuf[slot] = blk
        _out_copy(o_hbm, obuf, sems, i, slot).start(priority=1)

    @pl.when(i == _GRID - 1)
    def _():
        tailbuf[...] = blk[:, :_TAIL]
        tail = pltpu.make_async_copy(
            tailbuf,
            o_hbm.at[:, pl.ds((_GRID - 1) * NBLK, _TAIL)],
            tailsem,
        )
        tail.start(priority=0)
        for back in range(1, NBUF):
            st = _GRID - 1 - back
            _out_copy(o_hbm, obuf, sems, st, lax.rem(st, NBUF)).wait()
        tail.wait()


def kernel(inputs, emb_table, W, b):
    pooled = _pool_sc(inputs.reshape(-1), emb_table)
    return pl.pallas_call(
        _proj_tc,
        grid=(_GRID,),
        in_specs=[
            pl.BlockSpec((B, EMB), lambda i: (0, 0)),
            pl.BlockSpec((EMB, NBLK), lambda i: (0, i)),
            pl.BlockSpec((1, NBLK), lambda i: (0, i)),
        ],
        out_specs=pl.BlockSpec(memory_space=pl.ANY),
        out_shape=jax.ShapeDtypeStruct((B, VOCAB), jnp.float32),
        scratch_shapes=[
            pltpu.VMEM((NBUF, B, NBLK), jnp.float32),
            pltpu.VMEM((B, _TAIL), jnp.float32),
            pltpu.SemaphoreType.DMA((NBUF,)),
            pltpu.SemaphoreType.DMA,
        ],
    )(pooled, W, b.reshape(1, VOCAB))
